# baseline (speedup 1.0000x reference)
import functools

import jax
import jax.numpy as jnp
from jax import lax
from jax.experimental import pallas as pl
from jax.experimental.pallas import tpu as pltpu
from jax.experimental.pallas import tpu_sc as plsc

_NBINS = (128, 64, 32)
_SUM_H = sum(_NBINS)
_SUM_W = sum(n - 1 for n in _NBINS)
_N = 131072
_G_OI = 4096
_F = 512
_BL0 = 256

_NC = 2
_NS = 16
_NW = _NC * _NS
_NH = _N // 2
_PER_W = _NH // _NW
_C = 64
_CHUNKS = _PER_W // _C

_GB = 128


def _lane_cumsum(v, width):
    s = 1
    z = jnp.zeros_like(v)
    while s < width:
        v = v + jnp.concatenate([z[:, :s], v[:, :-s]], axis=-1)
        s *= 2
    return v


def _prep_body(goi_smem, hw_ref, ww_ref, out_ref, buf_uh, buf_uw, sem1, sem2):
    i = pl.program_id(0)

    def fire(g, carry):
        idx = goi_smem[i * _GB + g]
        pltpu.make_async_copy(hw_ref.at[pl.ds(idx, 1), :],
                              buf_uh.at[pl.ds(g, 1), :], sem1).start()
        pltpu.make_async_copy(ww_ref.at[pl.ds(idx, 1), :],
                              buf_uw.at[pl.ds(g, 1), :], sem2).start()
        return carry

    lax.fori_loop(0, _GB, fire, 0)

    def drain(g, carry):
        pltpu.make_async_copy(hw_ref.at[pl.ds(0, 1), :],
                              buf_uh.at[pl.ds(g, 1), :], sem1).wait()
        pltpu.make_async_copy(ww_ref.at[pl.ds(0, 1), :],
                              buf_uw.at[pl.ds(g, 1), :], sem2).wait()
        return carry

    lax.fori_loop(0, _GB, drain, 0)

    out_ref[:, 0:_SUM_H] = buf_uh[:, :]
    uw = buf_uw[:, :]
    hoff = 0
    woff = 0
    zero1 = jnp.zeros((_GB, 1), jnp.float32)
    one1 = jnp.ones((_GB, 1), jnp.float32)
    for K in _NBINS:
        Km1 = K - 1
        eu = jnp.exp(uw[:, woff:woff + Km1])
        cs = _lane_cumsum(eu, Km1)
        s = cs[:, Km1 - 1:Km1]
        binloc = jnp.concatenate(
            [zero1, cs[:, :Km1 - 1] / s, one1], axis=-1)
        out_ref[:, _BL0 + hoff:_BL0 + hoff + K] = binloc
        hoff += K
        woff += Km1


def _prep_call(goi, hw, ww):
    return pl.pallas_call(
        _prep_body,
        grid_spec=pltpu.PrefetchScalarGridSpec(
            num_scalar_prefetch=1,
            grid=(_G_OI // _GB,),
            in_specs=[
                pl.BlockSpec(memory_space=pl.ANY),
                pl.BlockSpec(memory_space=pl.ANY),
            ],
            out_specs=pl.BlockSpec((_GB, _F), lambda i, g: (i, 0)),
            scratch_shapes=[
                pltpu.VMEM((_GB, _SUM_H), jnp.float32),
                pltpu.VMEM((_GB, _SUM_W), jnp.float32),
                pltpu.SemaphoreType.DMA,
                pltpu.SemaphoreType.DMA,
            ],
        ),
        out_shape=jax.ShapeDtypeStruct((_G_OI, _F), jnp.float32),
    )(goi, hw, ww)


def _sc_gather(half, lgi_hbm, tab_hbm, rows_out,
               lgi_all, rows_v0, rows_v1, g0, g1, o0, o1):
    wid = lax.axis_index("s") * _NC + lax.axis_index("c")
    base = wid * _PER_W
    in_base = half * _NH + base
    npair = _CHUNKS // 2

    pltpu.sync_copy(lgi_hbm.at[pl.ds(in_base, _PER_W)], lgi_all)

    def idx(c):
        return lgi_all.at[pl.ds(c * _C, _C)]

    pltpu.async_copy(tab_hbm.at[idx(0)], rows_v0, g0)

    def pair(i, carry):
        c0 = 2 * i
        off0 = base + c0 * _C
        off1 = off0 + _C
        pltpu.make_async_copy(tab_hbm.at[idx(c0)], rows_v0, g0).wait()
        pltpu.async_copy(rows_v0, rows_out.at[pl.ds(off0, _C)], o0)

        @pl.when(i > 0)
        def _():
            pltpu.make_async_copy(rows_v1, rows_out.at[pl.ds(off1, _C)],
                                  o1).wait()

        pltpu.async_copy(tab_hbm.at[idx(c0 + 1)], rows_v1, g1)

        @pl.when(i < npair - 1)
        def _():
            pltpu.make_async_copy(rows_v0, rows_out.at[pl.ds(off0, _C)],
                                  o0).wait()
            pltpu.async_copy(tab_hbm.at[idx(c0 + 2)], rows_v0, g0)

        pltpu.make_async_copy(tab_hbm.at[idx(c0 + 1)], rows_v1, g1).wait()
        pltpu.async_copy(rows_v1, rows_out.at[pl.ds(off1, _C)], o1)
        return carry

    lax.fori_loop(0, npair, pair, 0)
    last1 = base + (_CHUNKS - 1) * _C
    pltpu.make_async_copy(rows_v0, rows_out.at[pl.ds(base, _C)], o0).wait()
    pltpu.make_async_copy(rows_v1, rows_out.at[pl.ds(last1, _C)], o1).wait()


@functools.lru_cache(maxsize=2)
def _sc_gather_call(half):
    return pl.kernel(
        functools.partial(_sc_gather, half),
        out_type=jax.ShapeDtypeStruct((_NH, _F), jnp.float32),
        mesh=plsc.VectorSubcoreMesh(core_axis_name="c", subcore_axis_name="s"),
        scratch_types=[
            pltpu.VMEM((_PER_W,), jnp.int32),
            pltpu.VMEM((_C, _F), jnp.float32),
            pltpu.VMEM((_C, _F), jnp.float32),
            pltpu.SemaphoreType.DMA,
            pltpu.SemaphoreType.DMA,
            pltpu.SemaphoreType.DMA,
            pltpu.SemaphoreType.DMA,
        ],
    )


_B = 512


def _tc_body(x_ref, delta_ref, fp_ref, out_ref, lad_ref):
    xcur = x_ref[:, 0:1]
    dprod = jnp.ones_like(xcur)
    zero1 = jnp.zeros((_B, 1), jnp.float32)
    one1 = jnp.ones((_B, 1), jnp.float32)
    hoff = 0
    for K in _NBINS:
        Km1 = K - 1
        bl = fp_ref[:, _BL0 + hoff:_BL0 + hoff + K]
        e = jnp.exp(fp_ref[:, hoff:hoff + K]
                    + delta_ref[:, hoff:hoff + K])
        blext = jnp.concatenate([bl[:, 1:K], one1], axis=-1)
        blshr = jnp.concatenate([zero1, bl[:, 0:Km1]], axis=-1)
        wp = blext - bl
        wq = bl - blshr
        r = e * (wp + wq)
        q = e * wq

        def red(v):
            return jnp.sum(v, axis=-1, keepdims=True)

        area = 0.5 * red(r)
        cnt = red((xcur >= bl).astype(jnp.float32))
        idx = jnp.clip(cnt.astype(jnp.int32) - 1, 0, Km1 - 1)
        iota = lax.broadcasted_iota(jnp.int32, (_B, K), 1)
        m_eq = iota == idx
        m_lt = iota < idx
        m_eq1 = iota == idx + 1
        zk = jnp.zeros((_B, K), jnp.float32)
        pref = 0.5 * red(jnp.where(m_lt, r, zk) + jnp.where(m_eq, q, zk))
        el = red(jnp.where(m_eq, e, zk))
        er = red(jnp.where(m_eq1, e, zk))
        bl_sel = red(jnp.where(m_eq, bl, zk))
        w_sel = red(jnp.where(m_eq, wp, zk))
        inv_area = 1.0 / area
        elh = el * inv_area
        erh = er * inv_area
        cdf_sel = pref * inv_area
        hdiff = erh - elh
        alpha = (xcur - bl_sel) / w_sel
        xnew = w_sel * alpha * (0.5 * hdiff * alpha + elh) + cdf_sel
        xcur = jnp.clip(xnew, 0.0, 1.0)
        dprod = dprod * (alpha * hdiff + elh)
        hoff += K
    out_ref[:, :] = xcur
    lad_ref[:, :] = jnp.log(dprod)


def _tc_call(x2, delta, fused_pts, half):
    grid = (_NH // _B,)
    boff = half * (_NH // _B)
    return pl.pallas_call(
        _tc_body,
        grid=grid,
        in_specs=[
            pl.BlockSpec((_B, 1), lambda i: (i + boff, 0)),
            pl.BlockSpec((_B, _SUM_H), lambda i: (i + boff, 0)),
            pl.BlockSpec((_B, _F), lambda i: (i, 0)),
        ],
        out_specs=[
            pl.BlockSpec((_B, 1), lambda i: (i, 0)),
            pl.BlockSpec((_B, 1), lambda i: (i, 0)),
        ],
        out_shape=[
            jax.ShapeDtypeStruct((_NH, 1), jnp.float32),
            jax.ShapeDtypeStruct((_NH, 1), jnp.float32),
        ],
    )(x2, delta, fused_pts)


def kernel(x, genes_oi, local_gene_ix, delta, heights_weight, widths_weight):
    fused_tab = _prep_call(genes_oi, heights_weight, widths_weight)
    x2 = x.reshape(_N, 1)
    pts0 = _sc_gather_call(0)(local_gene_ix, fused_tab)
    pts1 = _sc_gather_call(1)(local_gene_ix, fused_tab)
    out0, lad0 = _tc_call(x2, delta, pts0, 0)
    out1, lad1 = _tc_call(x2, delta, pts1, 1)
    out = jnp.concatenate([out0.reshape(_NH), out1.reshape(_NH)])
    lad = jnp.concatenate([lad0.reshape(_NH), lad1.reshape(_NH)])
    return out, lad


# speedup vs baseline: 1.0376x; 1.0376x over previous
import functools

import jax
import jax.numpy as jnp
from jax import lax
from jax.experimental import pallas as pl
from jax.experimental.pallas import tpu as pltpu
from jax.experimental.pallas import tpu_sc as plsc

_NBINS = (128, 64, 32)
_SUM_H = sum(_NBINS)
_SUM_W = sum(n - 1 for n in _NBINS)
_N = 131072
_G_OI = 4096
_F = 512
_BL0 = 256

_NC = 2
_NS = 16
_NW = _NC * _NS
_NH = _N
_PER_W = _NH // _NW
_C = 64
_CHUNKS = _PER_W // _C

_GB = 4096


def _lane_cumsum(v, width):
    s = 1
    z = jnp.zeros_like(v)
    while s < width:
        v = v + jnp.concatenate([z[:, :s], v[:, :-s]], axis=-1)
        s *= 2
    return v


def _prep_body(goi_smem, hw_ref, ww_ref, out_ref, buf_uh, buf_uw, sem1, sem2):
    def fire(g, carry):
        idx = goi_smem[g]
        pltpu.make_async_copy(hw_ref.at[pl.ds(idx, 1), :],
                              buf_uh.at[pl.ds(g, 1), :], sem1).start()
        pltpu.make_async_copy(ww_ref.at[pl.ds(idx, 1), :],
                              buf_uw.at[pl.ds(g, 1), :], sem2).start()
        return carry

    lax.fori_loop(0, _GB, fire, 0)
    pltpu.make_async_copy(hw_ref.at[pl.ds(0, _GB), :], buf_uh, sem1).wait()
    pltpu.make_async_copy(ww_ref.at[pl.ds(0, _GB), :], buf_uw, sem2).wait()

    out_ref[:, 0:_SUM_H] = buf_uh[:, :]
    uw = buf_uw[:, :]
    hoff = 0
    woff = 0
    zero1 = jnp.zeros((_GB, 1), jnp.float32)
    one1 = jnp.ones((_GB, 1), jnp.float32)
    for K in _NBINS:
        Km1 = K - 1
        eu = jnp.exp(uw[:, woff:woff + Km1])
        cs = _lane_cumsum(eu, Km1)
        s = cs[:, Km1 - 1:Km1]
        binloc = jnp.concatenate(
            [zero1, cs[:, :Km1 - 1] / s, one1], axis=-1)
        out_ref[:, _BL0 + hoff:_BL0 + hoff + K] = binloc
        hoff += K
        woff += Km1


def _prep_call(goi, hw, ww):
    return pl.pallas_call(
        _prep_body,
        grid_spec=pltpu.PrefetchScalarGridSpec(
            num_scalar_prefetch=1,
            grid=(_G_OI // _GB,),
            in_specs=[
                pl.BlockSpec(memory_space=pl.ANY),
                pl.BlockSpec(memory_space=pl.ANY),
            ],
            out_specs=pl.BlockSpec((_GB, _F), lambda i, g: (i, 0)),
            scratch_shapes=[
                pltpu.VMEM((_GB, _SUM_H), jnp.float32),
                pltpu.VMEM((_GB, _SUM_W), jnp.float32),
                pltpu.SemaphoreType.DMA,
                pltpu.SemaphoreType.DMA,
            ],
        ),
        out_shape=jax.ShapeDtypeStruct((_G_OI, _F), jnp.float32),
    )(goi, hw, ww)


def _sc_gather(half, lgi_hbm, tab_hbm, rows_out,
               lgi_all, rows_v0, rows_v1, g0, g1, o0, o1):
    wid = lax.axis_index("s") * _NC + lax.axis_index("c")
    base = wid * _PER_W
    in_base = half * _NH + base
    npair = _CHUNKS // 2

    pltpu.sync_copy(lgi_hbm.at[pl.ds(in_base, _PER_W)], lgi_all)

    def idx(c):
        return lgi_all.at[pl.ds(c * _C, _C)]

    pltpu.async_copy(tab_hbm.at[idx(0)], rows_v0, g0)

    def pair(i, carry):
        c0 = 2 * i
        off0 = base + c0 * _C
        off1 = off0 + _C
        pltpu.make_async_copy(tab_hbm.at[idx(c0)], rows_v0, g0).wait()
        pltpu.async_copy(rows_v0, rows_out.at[pl.ds(off0, _C)], o0)

        @pl.when(i > 0)
        def _():
            pltpu.make_async_copy(rows_v1, rows_out.at[pl.ds(off1, _C)],
                                  o1).wait()

        pltpu.async_copy(tab_hbm.at[idx(c0 + 1)], rows_v1, g1)

        @pl.when(i < npair - 1)
        def _():
            pltpu.make_async_copy(rows_v0, rows_out.at[pl.ds(off0, _C)],
                                  o0).wait()
            pltpu.async_copy(tab_hbm.at[idx(c0 + 2)], rows_v0, g0)

        pltpu.make_async_copy(tab_hbm.at[idx(c0 + 1)], rows_v1, g1).wait()
        pltpu.async_copy(rows_v1, rows_out.at[pl.ds(off1, _C)], o1)
        return carry

    lax.fori_loop(0, npair, pair, 0)
    last1 = base + (_CHUNKS - 1) * _C
    pltpu.make_async_copy(rows_v0, rows_out.at[pl.ds(base, _C)], o0).wait()
    pltpu.make_async_copy(rows_v1, rows_out.at[pl.ds(last1, _C)], o1).wait()


@functools.lru_cache(maxsize=2)
def _sc_gather_call(half):
    return pl.kernel(
        functools.partial(_sc_gather, half),
        out_type=jax.ShapeDtypeStruct((_NH, _F), jnp.float32),
        mesh=plsc.VectorSubcoreMesh(core_axis_name="c", subcore_axis_name="s"),
        scratch_types=[
            pltpu.VMEM((_PER_W,), jnp.int32),
            pltpu.VMEM((_C, _F), jnp.float32),
            pltpu.VMEM((_C, _F), jnp.float32),
            pltpu.SemaphoreType.DMA,
            pltpu.SemaphoreType.DMA,
            pltpu.SemaphoreType.DMA,
            pltpu.SemaphoreType.DMA,
        ],
    )


_B = 512


def _tc_body(x_ref, delta_ref, fp_ref, out_ref, lad_ref):
    xcur = x_ref[:, 0:1]
    dprod = jnp.ones_like(xcur)
    zero1 = jnp.zeros((_B, 1), jnp.float32)
    one1 = jnp.ones((_B, 1), jnp.float32)
    hoff = 0
    for K in _NBINS:
        Km1 = K - 1
        bl = fp_ref[:, _BL0 + hoff:_BL0 + hoff + K]
        e = jnp.exp(fp_ref[:, hoff:hoff + K]
                    + delta_ref[:, hoff:hoff + K])
        blext = jnp.concatenate([bl[:, 1:K], one1], axis=-1)
        blshr = jnp.concatenate([zero1, bl[:, 0:Km1]], axis=-1)
        wp = blext - bl
        wq = bl - blshr
        r = e * (wp + wq)
        q = e * wq

        def red(v):
            return jnp.sum(v, axis=-1, keepdims=True)

        area = 0.5 * red(r)
        cnt = red((xcur >= bl).astype(jnp.float32))
        idx = jnp.clip(cnt.astype(jnp.int32) - 1, 0, Km1 - 1)
        iota = lax.broadcasted_iota(jnp.int32, (_B, K), 1)
        m_eq = iota == idx
        m_lt = iota < idx
        m_eq1 = iota == idx + 1
        zk = jnp.zeros((_B, K), jnp.float32)
        pref = 0.5 * red(jnp.where(m_lt, r, zk) + jnp.where(m_eq, q, zk))
        el = red(jnp.where(m_eq, e, zk))
        er = red(jnp.where(m_eq1, e, zk))
        bl_sel = red(jnp.where(m_eq, bl, zk))
        w_sel = red(jnp.where(m_eq, wp, zk))
        inv_area = 1.0 / area
        elh = el * inv_area
        erh = er * inv_area
        cdf_sel = pref * inv_area
        hdiff = erh - elh
        alpha = (xcur - bl_sel) / w_sel
        xnew = w_sel * alpha * (0.5 * hdiff * alpha + elh) + cdf_sel
        xcur = jnp.clip(xnew, 0.0, 1.0)
        dprod = dprod * (alpha * hdiff + elh)
        hoff += K
    out_ref[:, :] = xcur
    lad_ref[:, :] = jnp.log(dprod)


def _tc_call(x2, delta, fused_pts, half):
    grid = (_NH // _B,)
    boff = half * (_NH // _B)
    return pl.pallas_call(
        _tc_body,
        grid=grid,
        in_specs=[
            pl.BlockSpec((_B, 1), lambda i: (i + boff, 0)),
            pl.BlockSpec((_B, _SUM_H), lambda i: (i + boff, 0)),
            pl.BlockSpec((_B, _F), lambda i: (i, 0)),
        ],
        out_specs=[
            pl.BlockSpec((_B, 1), lambda i: (i, 0)),
            pl.BlockSpec((_B, 1), lambda i: (i, 0)),
        ],
        out_shape=[
            jax.ShapeDtypeStruct((_NH, 1), jnp.float32),
            jax.ShapeDtypeStruct((_NH, 1), jnp.float32),
        ],
    )(x2, delta, fused_pts)


def kernel(x, genes_oi, local_gene_ix, delta, heights_weight, widths_weight):
    fused_tab = _prep_call(genes_oi, heights_weight, widths_weight)
    x2 = x.reshape(_N, 1)
    pts0 = _sc_gather_call(0)(local_gene_ix, fused_tab)
    out0, lad0 = _tc_call(x2, delta, pts0, 0)
    return out0.reshape(_N), lad0.reshape(_N)
